# baseline (speedup 1.0000x reference)
import functools

import jax
import jax.numpy as jnp
from jax.experimental import pallas as pl
from jax.experimental.pallas import tpu as pltpu

_NUM_LAYERS = 7
_HALF = 4
_N = 128
_G = 32

_dot = functools.partial(jax.lax.dot_general,
                         preferred_element_type=jnp.float32)


def _nn(a, b):
    return _dot(a, b, (((1,), (0,)), ((), ())))


def _nt(a, b):
    return _dot(a, b, (((1,), (1,)), ((), ())))


def kernel(nodes, adj, W0, W1, W2, W3, W4, W5, W6,
           b0, b1, b2, b3, b4, b5, b6, Wf1, bf1, Wf2, bf2):
    B, N, F0 = nodes.shape
    Ws = (W0, W1, W2, W3, W4, W5, W6)
    bs = (b0, b1, b2, b3, b4, b5, b6)

    feat_dims = [F0] + [W.shape[1] for W in Ws]
    wlist = []
    for i in range(_NUM_LAYERS):
        d = Ws[i].shape[0] // 2
        Wt, Wb = Ws[i][:d], Ws[i][d:]
        if i < _HALF:
            wlist += [Wt.T, Wb.T]
        else:
            hd = feat_dims[i]
            wlist += [Wt[:hd].T, Wt[hd:].T, Wb[:hd].T, Wb[hd:].T]
        wlist.append(bs[i].reshape(-1, 1))
    k = Wf1.shape[0] // 4
    wlist += [Wf1[:k], Wf1[k:2 * k], Wf1[2 * k:3 * k], Wf1[3 * k:],
              bf1.reshape(1, -1), Wf2, bf2.reshape(1, -1)]

    offs, shapes, r = [], [], 0
    for a in wlist:
        offs.append(r)
        shapes.append(a.shape)
        r += -(-a.shape[0] // 8) * 8
    total_rows = -(-r // 8) * 8
    wpack = jnp.zeros((total_rows, 128), jnp.float32)
    for a, off in zip(wlist, offs):
        wpack = wpack.at[off:off + a.shape[0], :a.shape[1]].set(a)

    def body(nodes_hbm, adj_hbm, wpack_ref, out_ref,
             nbuf, abuf, nsem, asem):
        i = pl.program_id(0)
        nsteps = pl.num_programs(0)
        slot = jax.lax.rem(i, 2)
        nxt = jax.lax.rem(i + 1, 2)

        def start(step, sl):
            pltpu.make_async_copy(
                adj_hbm.at[pl.ds(step * _G, _G)], abuf.at[sl],
                asem.at[sl]).start()
            pltpu.make_async_copy(
                nodes_hbm.at[pl.ds(step * _G, _G)], nbuf.at[sl],
                nsem.at[sl]).start()

        @pl.when(i == 0)
        def _():
            start(0, 0)

        @pl.when(i + 1 < nsteps)
        def _():
            start(i + 1, nxt)

        pltpu.make_async_copy(
            adj_hbm.at[pl.ds(i * _G, _G)], abuf.at[slot],
            asem.at[slot]).wait()
        pltpu.make_async_copy(
            nodes_hbm.at[pl.ds(i * _G, _G)], nbuf.at[slot],
            nsem.at[slot]).wait()

        adj_ref = abuf.at[slot]
        nodes_blk = nbuf[slot]

        wrefs = [wpack_ref[off:off + s[0], :s[1]]
                 for off, s in zip(offs, shapes)]
        idx = 0
        layers = []
        for li in range(_NUM_LAYERS):
            n = 3 if li < _HALF else 5
            layers.append(tuple(wrefs[idx:idx + n]))
            idx += n
        Wf1a, Wf1b, Wf1c, Wf1d, bf1r, Wf2r, bf2r = wrefs[idx:]

        ones_row = jnp.ones((1, _N), jnp.float32)
        invdegs = []
        for g in range(_G):
            deg = _nt(ones_row, adj_ref[g])
            invdegs.append(1.0 / jnp.maximum(deg, 1e-6))

        nds = nodes_blk.reshape(_G * _N, F0)

        hT = None
        outs = []
        for li in range(_NUM_LAYERS):
            if li < _HALF:
                WtT, WbT, bT = layers[li]
                if li == 0:
                    qT = _nt(WtT, nds)
                    pT = _nt(WbT, nds)
                else:
                    qT = _nn(WtT, hT)
                    pT = _nn(WbT, hT)
            else:
                WthT, WtsT, WbhT, WbsT, bT = layers[li]
                sT = outs[_NUM_LAYERS - 1 - li]
                qT = _nn(WthT, hT) + _nn(WtsT, sT)
                pT = _nn(WbhT, hT) + _nn(WbsT, sT)
            nbT = jnp.concatenate(
                [_nt(pT[:, g * _N:(g + 1) * _N], adj_ref[g]) * invdegs[g]
                 for g in range(_G)], axis=1)
            hT = jax.nn.relu(qT + nbT + bT)
            outs.append(hT)

        mxs, mns, sms = [], [], []
        for g in range(_G):
            hg = hT[:, g * _N:(g + 1) * _N]
            mxs.append(jnp.max(hg, axis=1, keepdims=True))
            mns.append(jnp.min(hg, axis=1, keepdims=True))
            sms.append(jnp.sum(hg, axis=1, keepdims=True))
        MXT = jnp.concatenate(mxs, axis=1)
        MNT = jnp.concatenate(mns, axis=1)
        SMT = jnp.concatenate(sms, axis=1)
        AVT = SMT * (1.0 / _N)

        rr = jax.lax.broadcasted_iota(jnp.int32, (_G, _G), 0)
        cc = jax.lax.broadcasted_iota(jnp.int32, (_G, _G), 1)
        eye = (rr == cc).astype(jnp.float32)
        MX = _nt(eye, MXT)
        MN = _nt(eye, MNT)
        AV = _nt(eye, AVT)
        SM = _nt(eye, SMT)

        hid = jax.nn.relu(
            _nn(MX, Wf1a) + _nn(MN, Wf1b) + _nn(AV, Wf1c) + _nn(SM, Wf1d)
            + bf1r)
        out_ref[...] = _nn(hid, Wf2r) + bf2r

    grid = (B // _G,)
    out = pl.pallas_call(
        body,
        grid=grid,
        in_specs=[
            pl.BlockSpec(memory_space=pl.ANY),
            pl.BlockSpec(memory_space=pl.ANY),
            pl.BlockSpec(wpack.shape, lambda i: (0, 0)),
        ],
        out_specs=pl.BlockSpec((_G, Wf2.shape[1]), lambda i: (i, 0)),
        out_shape=jax.ShapeDtypeStruct((B, Wf2.shape[1]), jnp.float32),
        scratch_shapes=[
            pltpu.VMEM((2, _G, _N, F0), jnp.float32),
            pltpu.VMEM((2, _G, _N, _N), jnp.float32),
            pltpu.SemaphoreType.DMA((2,)),
            pltpu.SemaphoreType.DMA((2,)),
        ],
    )(nodes, adj, wpack)
    return out


# speedup vs baseline: 1.1258x; 1.1258x over previous
import functools

import jax
import jax.numpy as jnp
from jax.experimental import pallas as pl
from jax.experimental.pallas import tpu as pltpu

_NUM_LAYERS = 7
_HALF = 4
_N = 128
_G = 64

_dot = functools.partial(jax.lax.dot_general,
                         preferred_element_type=jnp.float32)


def _nn(a, b):
    return _dot(a, b, (((1,), (0,)), ((), ())))


def _nt(a, b):
    return _dot(a, b, (((1,), (1,)), ((), ())))


def _body(nodes_ref, adj_ref, *refs):
    out_ref = refs[-1]
    wrefs = [r[...] for r in refs[:-1]]

    idx = 0
    layers = []
    for i in range(_NUM_LAYERS):
        n = 3 if i < _HALF else 5
        layers.append(tuple(wrefs[idx:idx + n]))
        idx += n
    Wf1a, Wf1b, Wf1c, Wf1d, bf1, Wf2, bf2 = wrefs[idx:]

    ones_row = jnp.ones((1, _N), jnp.float32)
    invdegs = []
    for g in range(_G):
        deg = _nt(ones_row, adj_ref[g])
        invdegs.append(1.0 / jnp.maximum(deg, 1e-6))

    nds = nodes_ref[...].reshape(_G * _N, nodes_ref.shape[2])

    hT = None
    outs = []
    for i in range(_NUM_LAYERS):
        if i < _HALF:
            WtT, WbT, bT = layers[i]
            if i == 0:
                qT = _nt(WtT, nds)
                pT = _nt(WbT, nds)
            else:
                qT = _nn(WtT, hT)
                pT = _nn(WbT, hT)
        else:
            WthT, WtsT, WbhT, WbsT, bT = layers[i]
            sT = outs[_NUM_LAYERS - 1 - i]
            qT = _nn(WthT, hT) + _nn(WtsT, sT)
            pT = _nn(WbhT, hT) + _nn(WbsT, sT)
        nbT = jnp.concatenate(
            [_nt(pT[:, g * _N:(g + 1) * _N], adj_ref[g]) * invdegs[g]
             for g in range(_G)], axis=1)
        hT = jax.nn.relu(qT + nbT + bT)
        outs.append(hT)

    mxs, mns, sms = [], [], []
    for g in range(_G):
        hg = hT[:, g * _N:(g + 1) * _N]
        mxs.append(jnp.max(hg, axis=1, keepdims=True))
        mns.append(jnp.min(hg, axis=1, keepdims=True))
        sms.append(jnp.sum(hg, axis=1, keepdims=True))
    MXT = jnp.concatenate(mxs, axis=1)
    MNT = jnp.concatenate(mns, axis=1)
    SMT = jnp.concatenate(sms, axis=1)
    AVT = SMT * (1.0 / _N)

    rr = jax.lax.broadcasted_iota(jnp.int32, (_G, _G), 0)
    cc = jax.lax.broadcasted_iota(jnp.int32, (_G, _G), 1)
    eye = (rr == cc).astype(jnp.float32)
    MX = _nt(eye, MXT)
    MN = _nt(eye, MNT)
    AV = _nt(eye, AVT)
    SM = _nt(eye, SMT)

    hid = jax.nn.relu(
        _nn(MX, Wf1a) + _nn(MN, Wf1b) + _nn(AV, Wf1c) + _nn(SM, Wf1d) + bf1)
    out_ref[...] = _nn(hid, Wf2) + bf2


def kernel(nodes, adj, W0, W1, W2, W3, W4, W5, W6,
           b0, b1, b2, b3, b4, b5, b6, Wf1, bf1, Wf2, bf2):
    B, N, F0 = nodes.shape
    Ws = (W0, W1, W2, W3, W4, W5, W6)
    bs = (b0, b1, b2, b3, b4, b5, b6)

    feat_dims = [F0] + [W.shape[1] for W in Ws]
    wargs = []
    for i in range(_NUM_LAYERS):
        d = Ws[i].shape[0] // 2
        Wt, Wb = Ws[i][:d], Ws[i][d:]
        if i < _HALF:
            wargs += [Wt.T, Wb.T]
        else:
            hd = feat_dims[i]
            wargs += [Wt[:hd].T, Wt[hd:].T, Wb[:hd].T, Wb[hd:].T]
        wargs.append(bs[i].reshape(-1, 1))
    k = Wf1.shape[0] // 4
    wargs += [Wf1[:k], Wf1[k:2 * k], Wf1[2 * k:3 * k], Wf1[3 * k:],
              bf1.reshape(1, -1), Wf2, bf2.reshape(1, -1)]

    grid = (B // _G,)
    const_spec = lambda a: pl.BlockSpec(a.shape, lambda i: (0,) * a.ndim)
    in_specs = [
        pl.BlockSpec((_G, N, F0), lambda i: (i, 0, 0)),
        pl.BlockSpec((_G, N, N), lambda i: (i, 0, 0)),
    ] + [const_spec(a) for a in wargs]

    out = pl.pallas_call(
        _body,
        grid=grid,
        in_specs=in_specs,
        out_specs=pl.BlockSpec((_G, Wf2.shape[1]), lambda i: (i, 0)),
        out_shape=jax.ShapeDtypeStruct((B, Wf2.shape[1]), jnp.float32),
        compiler_params=pltpu.CompilerParams(
            dimension_semantics=("parallel",)),
    )(nodes, adj, *wargs)
    return out


# speedup vs baseline: 1.1497x; 1.0212x over previous
import functools

import jax
import jax.numpy as jnp
from jax.experimental import pallas as pl
from jax.experimental.pallas import tpu as pltpu

_NUM_LAYERS = 7
_HALF = 4
_N = 128
_G = 128

_dot = functools.partial(jax.lax.dot_general,
                         preferred_element_type=jnp.float32)


def _nn(a, b):
    return _dot(a, b, (((1,), (0,)), ((), ())))


def _nt(a, b):
    return _dot(a, b, (((1,), (1,)), ((), ())))


def _body(nodes_ref, adj_ref, *refs):
    out_ref = refs[-1]
    wrefs = [r[...] for r in refs[:-1]]

    idx = 0
    layers = []
    for i in range(_NUM_LAYERS):
        n = 3 if i < _HALF else 5
        layers.append(tuple(wrefs[idx:idx + n]))
        idx += n
    Wf1a, Wf1b, Wf1c, Wf1d, bf1, Wf2, bf2 = wrefs[idx:]

    ones_row = jnp.ones((1, _N), jnp.float32)
    invdegs = []
    for g in range(_G):
        deg = _nt(ones_row, adj_ref[g])
        invdegs.append(1.0 / jnp.maximum(deg, 1e-6))

    nds = nodes_ref[...].reshape(_G * _N, nodes_ref.shape[2])

    hT = None
    outs = []
    for i in range(_NUM_LAYERS):
        if i < _HALF:
            WtT, WbT, bT = layers[i]
            if i == 0:
                qT = _nt(WtT, nds)
                pT = _nt(WbT, nds)
            else:
                qT = _nn(WtT, hT)
                pT = _nn(WbT, hT)
        else:
            WthT, WtsT, WbhT, WbsT, bT = layers[i]
            sT = outs[_NUM_LAYERS - 1 - i]
            qT = _nn(WthT, hT) + _nn(WtsT, sT)
            pT = _nn(WbhT, hT) + _nn(WbsT, sT)
        nbT = jnp.concatenate(
            [_nt(pT[:, g * _N:(g + 1) * _N], adj_ref[g]) * invdegs[g]
             for g in range(_G)], axis=1)
        hT = jax.nn.relu(qT + nbT + bT)
        outs.append(hT)

    mxs, mns, sms = [], [], []
    for g in range(_G):
        hg = hT[:, g * _N:(g + 1) * _N]
        mxs.append(jnp.max(hg, axis=1, keepdims=True))
        mns.append(jnp.min(hg, axis=1, keepdims=True))
        sms.append(jnp.sum(hg, axis=1, keepdims=True))
    MXT = jnp.concatenate(mxs, axis=1)
    MNT = jnp.concatenate(mns, axis=1)
    SMT = jnp.concatenate(sms, axis=1)
    AVT = SMT * (1.0 / _N)

    rr = jax.lax.broadcasted_iota(jnp.int32, (_G, _G), 0)
    cc = jax.lax.broadcasted_iota(jnp.int32, (_G, _G), 1)
    eye = (rr == cc).astype(jnp.float32)
    MX = _nt(eye, MXT)
    MN = _nt(eye, MNT)
    AV = _nt(eye, AVT)
    SM = _nt(eye, SMT)

    hid = jax.nn.relu(
        _nn(MX, Wf1a) + _nn(MN, Wf1b) + _nn(AV, Wf1c) + _nn(SM, Wf1d) + bf1)
    out_ref[...] = _nn(hid, Wf2) + bf2


def kernel(nodes, adj, W0, W1, W2, W3, W4, W5, W6,
           b0, b1, b2, b3, b4, b5, b6, Wf1, bf1, Wf2, bf2):
    B, N, F0 = nodes.shape
    Ws = (W0, W1, W2, W3, W4, W5, W6)
    bs = (b0, b1, b2, b3, b4, b5, b6)

    feat_dims = [F0] + [W.shape[1] for W in Ws]
    wargs = []
    for i in range(_NUM_LAYERS):
        d = Ws[i].shape[0] // 2
        Wt, Wb = Ws[i][:d], Ws[i][d:]
        if i < _HALF:
            wargs += [Wt.T, Wb.T]
        else:
            hd = feat_dims[i]
            wargs += [Wt[:hd].T, Wt[hd:].T, Wb[:hd].T, Wb[hd:].T]
        wargs.append(bs[i].reshape(-1, 1))
    k = Wf1.shape[0] // 4
    wargs += [Wf1[:k], Wf1[k:2 * k], Wf1[2 * k:3 * k], Wf1[3 * k:],
              bf1.reshape(1, -1), Wf2, bf2.reshape(1, -1)]

    grid = (B // _G,)
    const_spec = lambda a: pl.BlockSpec(a.shape, lambda i: (0,) * a.ndim)
    in_specs = [
        pl.BlockSpec((_G, N, F0), lambda i: (i, 0, 0)),
        pl.BlockSpec((_G, N, N), lambda i: (i, 0, 0)),
    ] + [const_spec(a) for a in wargs]

    out = pl.pallas_call(
        _body,
        grid=grid,
        in_specs=in_specs,
        out_specs=pl.BlockSpec((_G, Wf2.shape[1]), lambda i: (i, 0)),
        out_shape=jax.ShapeDtypeStruct((B, Wf2.shape[1]), jnp.float32),
        compiler_params=pltpu.CompilerParams(
            dimension_semantics=("parallel",)),
    )(nodes, adj, *wargs)
    return out
